# baseline (speedup 1.0000x reference)
import functools

import jax
import jax.numpy as jnp
from jax import lax
from jax.experimental import pallas as pl
from jax.experimental.pallas import tpu as pltpu
from jax.experimental.pallas import tpu_sc as plsc

_N, _T, _C = 4096, 512, 16
_NC, _NS, _L = 2, 16, 16
_NW = _NC * _NS
_ROWS_PER_W = _N // _NW
_CHUNK_ROWS = 8
_CHUNK_ELEMS = _CHUNK_ROWS * _T
_CHUNK_FLOATS = _CHUNK_ELEMS * _C
_NCHUNKS = _ROWS_PER_W // _CHUNK_ROWS
_GROUPS = _CHUNK_ELEMS // _L
_GPR = _T // _L

_A0, _A1, _A2 = 0.4, 0.3, 0.3
_INV_T = 1.0 / _T


def _sc_body(logp_hbm, y_hbm, tl_hbm, out_hbm, buf, ybuf, tlbuf, accbuf):
    cid = lax.axis_index("c")
    sid = lax.axis_index("s")
    wid = sid * _NC + cid
    base_e0 = wid * (_ROWS_PER_W * _T)

    lanes = lax.iota(jnp.int32, _L)
    lanes_f = lanes.astype(jnp.float32)

    def chunk_body(ci, acc):
        base_e = base_e0 + ci * _CHUNK_ELEMS
        pltpu.sync_copy(logp_hbm.at[pl.ds(base_e * _C, _CHUNK_FLOATS)], buf)
        pltpu.sync_copy(y_hbm.at[pl.ds(base_e, _CHUNK_ELEMS)], ybuf)
        pltpu.sync_copy(tl_hbm.at[pl.ds(base_e, _CHUNK_ELEMS)], tlbuf)

        def group_body(g, acc):
            e0 = g * _L
            r = lax.shift_right_logical(g, 5)
            tbase = lax.shift_left(g & (_GPR - 1), 4)
            y = ybuf[pl.ds(e0, _L)]
            tl = tlbuf[pl.ds(e0, _L)]
            t = tbase + lanes
            idx = r * (_T * _C) + lax.shift_left(t, 4) + y
            v = plsc.load_gather(buf, [idx])
            p = jnp.exp(v)
            tf = (jnp.float32(tbase) + lanes_f) * _INV_T
            tlf = tl.astype(jnp.float32) * _INV_T
            a1 = 1.0 - tf
            w1 = a1 * (1.0 - tlf)
            w2 = (a1 * a1) * (tlf * tlf)
            return acc + (-_A0) * v - _A1 * (p * w1) - _A2 * ((1.0 - p) * w2)

        return lax.fori_loop(0, _GROUPS, group_body, acc)

    acc = lax.fori_loop(0, _NCHUNKS, chunk_body, jnp.zeros((_L,), jnp.float32))
    accbuf[...] = acc
    pltpu.sync_copy(accbuf, out_hbm.at[wid])


@jax.jit
def _partials(logp_flat, y_flat, tl_flat):
    mesh = plsc.VectorSubcoreMesh(core_axis_name="c", subcore_axis_name="s")
    return pl.kernel(
        _sc_body,
        out_type=jax.ShapeDtypeStruct((_NW, _L), jnp.float32),
        mesh=mesh,
        scratch_types=[
            pltpu.VMEM((_CHUNK_FLOATS,), jnp.float32),
            pltpu.VMEM((_CHUNK_ELEMS,), jnp.int32),
            pltpu.VMEM((_CHUNK_ELEMS,), jnp.int32),
            pltpu.VMEM((_L,), jnp.float32),
        ],
        compiler_params=pltpu.CompilerParams(needs_layout_passes=False),
    )(logp_flat, y_flat, tl_flat)


def kernel(log_class_probabilities, timestamps_left, y_true):
    logp_flat = log_class_probabilities.reshape(-1)
    y_flat = y_true.reshape(-1)
    tl_flat = timestamps_left.reshape(-1)
    part = _partials(logp_flat, y_flat, tl_flat)
    return jnp.sum(part) * (1.0 / _N)


# speedup vs baseline: 2.6305x; 2.6305x over previous
import functools

import jax
import jax.numpy as jnp
from jax import lax
from jax.experimental import pallas as pl
from jax.experimental.pallas import tpu as pltpu
from jax.experimental.pallas import tpu_sc as plsc

_N, _T, _C = 4096, 512, 16
_NC, _NS, _L = 2, 16, 16
_NW = _NC * _NS
_ROWS_PER_W = _N // _NW
_CHUNK_ROWS = 8
_CHUNK_ELEMS = _CHUNK_ROWS * _T
_NCHUNKS = _ROWS_PER_W // _CHUNK_ROWS
_GROUPS = _CHUNK_ELEMS // _L
_GPR = _T // _L

_A0, _A1, _A2 = 0.4, 0.3, 0.3
_INV_T = 1.0 / _T


def _sc_body(logp_hbm, y_hbm, tl_hbm, out_hbm, buf, ybuf, tlbuf, accbuf):
    cid = lax.axis_index("c")
    sid = lax.axis_index("s")
    wid = sid * _NC + cid
    row0 = wid * _ROWS_PER_W

    lanes = lax.iota(jnp.int32, _L)
    lanes_f = lanes.astype(jnp.float32)

    def chunk_body(ci, acc):
        r0 = row0 + ci * _CHUNK_ROWS
        pltpu.sync_copy(logp_hbm.at[pl.ds(r0, _CHUNK_ROWS)], buf)
        pltpu.sync_copy(y_hbm.at[pl.ds(r0, _CHUNK_ROWS)], ybuf)
        pltpu.sync_copy(tl_hbm.at[pl.ds(r0, _CHUNK_ROWS)], tlbuf)

        def group_body(g, acc):
            r = lax.shift_right_logical(g, 5)
            tbase = lax.shift_left(g & (_GPR - 1), 4)
            y = ybuf[r, pl.ds(tbase, _L)]
            tl = tlbuf[r, pl.ds(tbase, _L)]
            t = tbase + lanes
            rv = jnp.full((_L,), r, dtype=jnp.int32)
            col = lax.shift_left(t, 4) + y
            v = plsc.load_gather(buf, [rv, col])
            p = jnp.exp(v)
            tf = (jnp.float32(tbase) + lanes_f) * _INV_T
            tlf = tl.astype(jnp.float32) * _INV_T
            a1 = 1.0 - tf
            w1 = a1 * (1.0 - tlf)
            w2 = (a1 * a1) * (tlf * tlf)
            return acc + (-_A0) * v - _A1 * (p * w1) - _A2 * ((1.0 - p) * w2)

        return lax.fori_loop(0, _GROUPS, group_body, acc)

    acc = lax.fori_loop(0, _NCHUNKS, chunk_body, jnp.zeros((_L,), jnp.float32))
    accbuf[...] = acc
    pltpu.sync_copy(accbuf, out_hbm.at[wid])


@jax.jit
def _partials(logp, y, tl):
    mesh = plsc.VectorSubcoreMesh(core_axis_name="c", subcore_axis_name="s")
    return pl.kernel(
        _sc_body,
        out_type=jax.ShapeDtypeStruct((_NW, _L), jnp.float32),
        mesh=mesh,
        scratch_types=[
            pltpu.VMEM((_CHUNK_ROWS, _T * _C), jnp.float32),
            pltpu.VMEM((_CHUNK_ROWS, _T), jnp.int32),
            pltpu.VMEM((_CHUNK_ROWS, _T), jnp.int32),
            pltpu.VMEM((_L,), jnp.float32),
        ],
        compiler_params=pltpu.CompilerParams(needs_layout_passes=False),
    )(logp, y, tl)


def kernel(log_class_probabilities, timestamps_left, y_true):
    logp2d = log_class_probabilities.reshape(_N, _T * _C)
    part = _partials(logp2d, y_true, timestamps_left)
    return jnp.sum(part) * (1.0 / _N)


# speedup vs baseline: 6.9852x; 2.6555x over previous
import functools

import jax
import jax.numpy as jnp
from jax import lax
from jax.experimental import pallas as pl
from jax.experimental.pallas import tpu as pltpu
from jax.experimental.pallas import tpu_sc as plsc

_N, _T, _C = 4096, 512, 16
_NC, _NS, _L = 2, 16, 16
_NW = _NC * _NS
_ROWS_PER_W = _N // _NW
_CHUNK_ROWS = 8
_CHUNK_ELEMS = _CHUNK_ROWS * _T
_NCHUNKS = _ROWS_PER_W // _CHUNK_ROWS
_GROUPS = _CHUNK_ELEMS // _L
_GPR = _T // _L

_A0, _A1, _A2 = 0.4, 0.3, 0.3
_INV_T = 1.0 / _T


def _sc_body(logp_hbm, y_hbm, tl_hbm, out_hbm, buf, ybuf, tlbuf, accbuf):
    cid = lax.axis_index("c")
    sid = lax.axis_index("s")
    wid = sid * _NC + cid
    row0 = wid * _ROWS_PER_W

    lanes = lax.iota(jnp.int32, _L)
    lanes_f = lanes.astype(jnp.float32)

    def chunk_body(ci, acc):
        r0 = row0 + ci * _CHUNK_ROWS
        pltpu.sync_copy(logp_hbm.at[pl.ds(r0, _CHUNK_ROWS)], buf)
        pltpu.sync_copy(y_hbm.at[pl.ds(r0, _CHUNK_ROWS)], ybuf)
        pltpu.sync_copy(tl_hbm.at[pl.ds(r0, _CHUNK_ROWS)], tlbuf)

        def group_body(g, acc):
            r = lax.shift_right_logical(g, 5)
            tbase = lax.shift_left(g & (_GPR - 1), 4)
            y = ybuf[r, pl.ds(tbase, _L)]
            tl = tlbuf[r, pl.ds(tbase, _L)]
            t = tbase + lanes
            rv = jnp.full((_L,), r, dtype=jnp.int32)
            v = plsc.load_gather(buf, [rv, y, t])
            p = jnp.exp(v)
            tf = (jnp.float32(tbase) + lanes_f) * _INV_T
            tlf = tl.astype(jnp.float32) * _INV_T
            a1 = 1.0 - tf
            w1 = a1 * (1.0 - tlf)
            w2 = (a1 * a1) * (tlf * tlf)
            return acc + (-_A0) * v - _A1 * (p * w1) - _A2 * ((1.0 - p) * w2)

        return lax.fori_loop(0, _GROUPS, group_body, acc)

    acc = lax.fori_loop(0, _NCHUNKS, chunk_body, jnp.zeros((_L,), jnp.float32))
    accbuf[...] = acc
    pltpu.sync_copy(accbuf, out_hbm.at[wid])


@jax.jit
def _partials(logp, y, tl):
    mesh = plsc.VectorSubcoreMesh(core_axis_name="c", subcore_axis_name="s")
    return pl.kernel(
        _sc_body,
        out_type=jax.ShapeDtypeStruct((_NW, _L), jnp.float32),
        mesh=mesh,
        scratch_types=[
            pltpu.VMEM((_CHUNK_ROWS, _C, _T), jnp.float32),
            pltpu.VMEM((_CHUNK_ROWS, _T), jnp.int32),
            pltpu.VMEM((_CHUNK_ROWS, _T), jnp.int32),
            pltpu.VMEM((_L,), jnp.float32),
        ],
        compiler_params=pltpu.CompilerParams(needs_layout_passes=False),
    )(logp, y, tl)


def kernel(log_class_probabilities, timestamps_left, y_true):
    logp_t = jnp.transpose(log_class_probabilities, (0, 2, 1))
    part = _partials(logp_t, y_true, timestamps_left)
    return jnp.sum(part) * (1.0 / _N)


# speedup vs baseline: 9.4811x; 1.3573x over previous
import functools

import jax
import jax.numpy as jnp
from jax import lax
from jax.experimental import pallas as pl
from jax.experimental.pallas import tpu as pltpu
from jax.experimental.pallas import tpu_sc as plsc

_N, _T, _C = 4096, 512, 16
_NC, _NS, _L = 2, 16, 16
_NW = _NC * _NS
_ROWS_PER_W = _N // _NW
_CHUNK_ROWS = 4
_CHUNK_ELEMS = _CHUNK_ROWS * _T
_NCHUNKS = _ROWS_PER_W // _CHUNK_ROWS
_GROUPS = _CHUNK_ELEMS // _L
_GPR = _T // _L

_A0, _A1, _A2 = 0.4, 0.3, 0.3
_INV_T = 1.0 / _T


def _sc_body(logp_hbm, y_hbm, tl_hbm, out_hbm,
             buf0, buf1, ybuf0, ybuf1, tlbuf0, tlbuf1, accbuf, sem0, sem1):
    cid = lax.axis_index("c")
    sid = lax.axis_index("s")
    wid = sid * _NC + cid
    row0 = wid * _ROWS_PER_W

    bufs = (buf0, buf1)
    ybufs = (ybuf0, ybuf1)
    tlbufs = (tlbuf0, tlbuf1)
    sems = (sem0, sem1)

    lanes = lax.iota(jnp.int32, _L)
    lanes_f = lanes.astype(jnp.float32)

    def start(ci):
        p = ci & 1
        r0 = row0 + ci * _CHUNK_ROWS
        d0 = pltpu.async_copy(logp_hbm.at[pl.ds(r0, _CHUNK_ROWS)], bufs[p], sems[p])
        d1 = pltpu.async_copy(y_hbm.at[pl.ds(r0, _CHUNK_ROWS)], ybufs[p], sems[p])
        d2 = pltpu.async_copy(tl_hbm.at[pl.ds(r0, _CHUNK_ROWS)], tlbufs[p], sems[p])
        return (d0, d1, d2)

    def process(ci, acc):
        p = ci & 1
        buf, ybuf, tlbuf = bufs[p], ybufs[p], tlbufs[p]

        def group_body(g, acc):
            r = lax.shift_right_logical(g, 5)
            tbase = lax.shift_left(g & (_GPR - 1), 4)
            y = ybuf[r, pl.ds(tbase, _L)]
            tl = tlbuf[r, pl.ds(tbase, _L)]
            t = tbase + lanes
            rv = jnp.full((_L,), r, dtype=jnp.int32)
            v = plsc.load_gather(buf, [rv, y, t])
            p_corr = jnp.exp(v)
            tf = (jnp.float32(tbase) + lanes_f) * _INV_T
            tlf = tl.astype(jnp.float32) * _INV_T
            a1 = 1.0 - tf
            w1 = a1 * (1.0 - tlf)
            w2 = (a1 * a1) * (tlf * tlf)
            return acc + (-_A0) * v - _A1 * (p_corr * w1) - _A2 * ((1.0 - p_corr) * w2)

        return lax.fori_loop(0, _GROUPS, group_body, acc)

    acc = jnp.zeros((_L,), jnp.float32)
    inflight = start(0)
    for ci in range(_NCHUNKS):
        nxt = start(ci + 1) if ci + 1 < _NCHUNKS else None
        for d in inflight:
            d.wait()
        acc = process(ci, acc)
        inflight = nxt

    accbuf[...] = acc
    pltpu.sync_copy(accbuf, out_hbm.at[wid])


@jax.jit
def _partials(logp, y, tl):
    mesh = plsc.VectorSubcoreMesh(core_axis_name="c", subcore_axis_name="s")
    return pl.kernel(
        _sc_body,
        out_type=jax.ShapeDtypeStruct((_NW, _L), jnp.float32),
        mesh=mesh,
        scratch_types=[
            pltpu.VMEM((_CHUNK_ROWS, _C, _T), jnp.float32),
            pltpu.VMEM((_CHUNK_ROWS, _C, _T), jnp.float32),
            pltpu.VMEM((_CHUNK_ROWS, _T), jnp.int32),
            pltpu.VMEM((_CHUNK_ROWS, _T), jnp.int32),
            pltpu.VMEM((_CHUNK_ROWS, _T), jnp.int32),
            pltpu.VMEM((_CHUNK_ROWS, _T), jnp.int32),
            pltpu.VMEM((_L,), jnp.float32),
            pltpu.SemaphoreType.DMA,
            pltpu.SemaphoreType.DMA,
        ],
        compiler_params=pltpu.CompilerParams(needs_layout_passes=False),
    )(logp, y, tl)


def kernel(log_class_probabilities, timestamps_left, y_true):
    logp_t = jnp.transpose(log_class_probabilities, (0, 2, 1))
    part = _partials(logp_t, y_true, timestamps_left)
    return jnp.sum(part) * (1.0 / _N)
